# baseline (speedup 1.0000x reference)
import functools

import jax
import jax.numpy as jnp
from jax import lax
from jax.experimental import pallas as pl
from jax.experimental.pallas import tpu as pltpu
from jax.experimental.pallas import tpu_sc as plsc

_D = 64
_SCALE = 8.0
_NC = 2
_NS = 16
_NW = _NC * _NS
_BW = 128
_NBUF = 3
_PF = 2


@functools.lru_cache(maxsize=None)
def _build(batch, seq):
    assert batch == _BW * _NW
    mesh = plsc.VectorSubcoreMesh(core_axis_name="c", subcore_axis_name="s")

    @functools.partial(
        pl.kernel,
        mesh=mesh,
        out_type=jax.ShapeDtypeStruct((seq, _D, batch), jnp.float32),
        compiler_params=pltpu.CompilerParams(needs_layout_passes=False),
        scratch_types=[
            pltpu.VMEM((_BW * seq,), jnp.int32),
            pltpu.VMEM((seq, _BW), jnp.int32),
            pltpu.VMEM((_NBUF, _BW, 128), jnp.float32),
            pltpu.VMEM((_NBUF, _D, _BW), jnp.float32),
            pltpu.SemaphoreType.DMA((_NBUF,)),
            pltpu.SemaphoreType.DMA((_NBUF,)),
        ],
    )
    def gather_scale(idx_hbm, table_hbm, out_hbm, xb_v, idx_v, rows_v, tp_v,
                     g_sem, s_sem):
        wid = lax.axis_index("s") * _NC + lax.axis_index("c")
        b0 = wid * _BW

        lane = lax.iota(jnp.int32, 16)

        pltpu.sync_copy(idx_hbm.at[pl.ds(b0 * seq, _BW * seq)], xb_v)

        def xt_body(s, carry):
            for jb in range(_BW // 16):
                src = plsc.load_gather(xb_v, [(lane + jb * 16) * seq + s])
                idx_v[s, pl.ds(jb * 16, 16)] = src
            return carry

        lax.fori_loop(0, seq, xt_body, 0)

        def gather_start(s, b):
            pltpu.async_copy(table_hbm.at[idx_v.at[s]], rows_v.at[b],
                             g_sem.at[b])

        def gather_wait(b):
            pltpu.make_async_copy(table_hbm.at[idx_v.at[0]], rows_v.at[b],
                                  g_sem.at[b]).wait()

        def scatter_start(s, b):
            pltpu.async_copy(tp_v.at[b], out_hbm.at[s, :, pl.ds(b0, _BW)],
                             s_sem.at[b])

        def scatter_wait(b):
            pltpu.make_async_copy(tp_v.at[b], out_hbm.at[0, :, pl.ds(b0, _BW)],
                                  s_sem.at[b]).wait()

        for b in range(_PF):
            gather_start(b, b)

        def chunk_case(b, s):
            gather_wait(b)

            bb = jnp.full((16,), b, jnp.int32)

            def tr_body(d, c2):
                dd = jnp.full((16,), 0, jnp.int32) + d
                for jb in range(_BW // 16):
                    src = plsc.load_gather(rows_v, [bb, lane + jb * 16, dd])
                    tp_v[b, d, pl.ds(jb * 16, 16)] = src * _SCALE
                return c2

            lax.fori_loop(0, _D, tr_body, 0)
            scatter_start(s, b)

            sp = s + _PF
            bp = (b + _PF) % _NBUF

            @pl.when(sp < seq)
            def _():
                @pl.when(sp >= _NBUF)
                def _():
                    scatter_wait(bp)

                gather_start(sp, bp)

        def chunk(s, carry):
            lax.switch(
                lax.rem(s, _NBUF),
                [functools.partial(chunk_case, b, s) for b in range(_NBUF)],
            )
            return carry

        lax.fori_loop(0, seq, chunk, 0)

        for b in range(_NBUF):
            scatter_wait(b)

    return gather_scale


def kernel(x, embedding):
    batch, seq = x.shape
    idx = x.reshape(-1).astype(jnp.int32)
    tab_p = jnp.pad(embedding, ((0, 0), (0, 128 - _D)))
    out_t = _build(batch, seq)(idx, tab_p)
    return jnp.transpose(out_t, (2, 0, 1))


# speedup vs baseline: 1.6523x; 1.6523x over previous
import functools

import jax
import jax.numpy as jnp
from jax import lax
from jax.experimental import pallas as pl
from jax.experimental.pallas import tpu as pltpu
from jax.experimental.pallas import tpu_sc as plsc

_D = 64
_SCALE = 8.0
_NC = 2
_NS = 16
_NW = _NC * _NS
_BW = 128
_K = 40
_CPB = 5
_NBUF = 4
_PF = 2


@functools.lru_cache(maxsize=None)
def _build(batch, seq):
    assert batch == _BW * _NW and seq == _K * _CPB
    n_chunks = _BW * _CPB
    mesh = plsc.VectorSubcoreMesh(core_axis_name="c", subcore_axis_name="s")

    @functools.partial(
        pl.kernel,
        mesh=mesh,
        out_type=jax.ShapeDtypeStruct((batch, seq, 128), jnp.float32),
        compiler_params=pltpu.CompilerParams(needs_layout_passes=False),
        scratch_types=[
            pltpu.VMEM((_BW * seq,), jnp.int32),
            pltpu.VMEM((_NBUF, _K, 128), jnp.float32),
            pltpu.SemaphoreType.DMA((_NBUF,)),
            pltpu.SemaphoreType.DMA((_NBUF,)),
        ],
    )
    def gather_scale(idx_hbm, table_hbm, out_hbm, xb_v, rows_v, g_sem, s_sem):
        wid = lax.axis_index("s") * _NC + lax.axis_index("c")
        b0 = wid * _BW

        pltpu.sync_copy(idx_hbm.at[pl.ds(b0 * seq, _BW * seq)], xb_v)

        def gather_start(c, b):
            pltpu.async_copy(table_hbm.at[xb_v.at[pl.ds(c * _K, _K)]],
                             rows_v.at[b], g_sem.at[b])

        def gather_wait(b):
            pltpu.make_async_copy(table_hbm.at[xb_v.at[pl.ds(0, _K)]],
                                  rows_v.at[b], g_sem.at[b]).wait()

        def scatter_start(c, b):
            bi = b0 + lax.div(c, _CPB)
            s0 = lax.rem(c, _CPB) * _K
            pltpu.async_copy(rows_v.at[b], out_hbm.at[bi, pl.ds(s0, _K)],
                             s_sem.at[b])

        def scatter_wait(b):
            pltpu.make_async_copy(rows_v.at[b], out_hbm.at[0, pl.ds(0, _K)],
                                  s_sem.at[b]).wait()

        for b in range(_PF):
            gather_start(b, b)

        def chunk_case(b, c):
            gather_wait(b)

            def sc_body(r, c2):
                for u in range(128 // 16):
                    rows_v[b, r, pl.ds(u * 16, 16)] = (
                        rows_v[b, r, pl.ds(u * 16, 16)] * _SCALE
                    )
                return c2

            lax.fori_loop(0, _K, sc_body, 0)
            scatter_start(c, b)

            cp = c + _PF
            bp = (b + _PF) % _NBUF

            @pl.when(cp < n_chunks)
            def _():
                @pl.when(cp >= _NBUF)
                def _():
                    scatter_wait(bp)

                gather_start(cp, bp)

        def chunk(c, carry):
            lax.switch(
                lax.rem(c, _NBUF),
                [functools.partial(chunk_case, b, c) for b in range(_NBUF)],
            )
            return carry

        lax.fori_loop(0, n_chunks, chunk, 0)

        for b in range(_NBUF):
            scatter_wait(b)

    return gather_scale


def kernel(x, embedding):
    batch, seq = x.shape
    idx = x.reshape(-1).astype(jnp.int32)
    tab_p = jnp.pad(embedding, ((0, 0), (0, 128 - _D)))
    out_p = _build(batch, seq)(idx, tab_p)
    return out_p[:, :, :_D]


# speedup vs baseline: 1.6826x; 1.0183x over previous
import functools

import jax
import jax.numpy as jnp
from jax import lax
from jax.experimental import pallas as pl
from jax.experimental.pallas import tpu as pltpu
from jax.experimental.pallas import tpu_sc as plsc

_D = 64
_SCALE = 8.0
_NC = 2
_NS = 16
_NW = _NC * _NS
_BW = 128
_K = 40
_CPB = 5
_NBUF = 4
_PF = 2


@functools.lru_cache(maxsize=None)
def _build(batch, seq):
    assert batch == _BW * _NW and seq == _K * _CPB
    n_chunks = _BW * _CPB
    mesh = plsc.VectorSubcoreMesh(core_axis_name="c", subcore_axis_name="s")

    @functools.partial(
        pl.kernel,
        mesh=mesh,
        out_type=jax.ShapeDtypeStruct((batch, seq, 128), jnp.float32),
        compiler_params=pltpu.CompilerParams(needs_layout_passes=False),
        scratch_types=[
            pltpu.VMEM((_BW * seq,), jnp.int32),
            pltpu.VMEM((_NBUF, _K, 128), jnp.float32),
            pltpu.SemaphoreType.DMA((_NBUF,)),
            pltpu.SemaphoreType.DMA((_NBUF,)),
        ],
    )
    def gather_scale(idx_hbm, table_hbm, out_hbm, xb_v, rows_v, g_sem, s_sem):
        wid = lax.axis_index("s") * _NC + lax.axis_index("c")
        b0 = wid * _BW

        pltpu.sync_copy(idx_hbm.at[pl.ds(b0 * seq, _BW * seq)], xb_v)

        def gather_start(c, b):
            pltpu.async_copy(table_hbm.at[xb_v.at[pl.ds(c * _K, _K)]],
                             rows_v.at[b], g_sem.at[b])

        def gather_wait(b):
            pltpu.make_async_copy(table_hbm.at[xb_v.at[pl.ds(0, _K)]],
                                  rows_v.at[b], g_sem.at[b]).wait()

        def scatter_start(c, b):
            bi = b0 + lax.div(c, _CPB)
            s0 = lax.rem(c, _CPB) * _K
            pltpu.async_copy(rows_v.at[b], out_hbm.at[bi, pl.ds(s0, _K)],
                             s_sem.at[b])

        def scatter_wait(b):
            pltpu.make_async_copy(rows_v.at[b], out_hbm.at[0, pl.ds(0, _K)],
                                  s_sem.at[b]).wait()

        for b in range(_PF):
            gather_start(b, b)

        def group(g, carry):
            for b in range(_NBUF):
                c = g * _NBUF + b
                gather_wait(b)

                def sc_body(r0, c2):
                    for ur in range(4):
                        r = r0 * 4 + ur
                        for u in range(_D // 16):
                            rows_v[b, r, pl.ds(u * 16, 16)] = (
                                rows_v[b, r, pl.ds(u * 16, 16)] * _SCALE
                            )
                    return c2

                lax.fori_loop(0, _K // 4, sc_body, 0)
                scatter_start(c, b)

                cp = c + _PF
                bp = (b + _PF) % _NBUF

                @pl.when(cp < n_chunks)
                def _():
                    @pl.when(cp >= _NBUF)
                    def _():
                        scatter_wait(bp)

                    gather_start(cp, bp)

            return carry

        lax.fori_loop(0, n_chunks // _NBUF, group, 0)

        for b in range(_NBUF):
            scatter_wait(b)

    return gather_scale


def kernel(x, embedding):
    batch, seq = x.shape
    idx = x.reshape(-1).astype(jnp.int32)
    tab_p = jnp.pad(embedding, ((0, 0), (0, 128 - _D)))
    out_p = _build(batch, seq)(idx, tab_p)
    return out_p[:, :, :_D]


# speedup vs baseline: 1.8718x; 1.1125x over previous
import functools

import jax
import jax.numpy as jnp
from jax import lax
from jax.experimental import pallas as pl
from jax.experimental.pallas import tpu as pltpu
from jax.experimental.pallas import tpu_sc as plsc

_D = 64
_SCALE = 8.0
_NC = 2
_NS = 16
_NW = _NC * _NS
_BW = 128
_K = 128
_NBUF = 4
_PF = 2


@functools.lru_cache(maxsize=None)
def _build(batch, seq):
    assert batch == _BW * _NW
    n_chunks = _BW * seq // _K
    mesh = plsc.VectorSubcoreMesh(core_axis_name="c", subcore_axis_name="s")

    @functools.partial(
        pl.kernel,
        mesh=mesh,
        out_type=jax.ShapeDtypeStruct((batch * seq, 128), jnp.float32),
        compiler_params=pltpu.CompilerParams(needs_layout_passes=False),
        scratch_types=[
            pltpu.VMEM((_BW * seq,), jnp.int32),
            pltpu.VMEM((_NBUF, _K, 128), jnp.float32),
            pltpu.SemaphoreType.DMA((_NBUF,)),
            pltpu.SemaphoreType.DMA((_NBUF,)),
        ],
    )
    def gather_scale(idx_hbm, table_hbm, out_hbm, xb_v, rows_v, g_sem, s_sem):
        wid = lax.axis_index("s") * _NC + lax.axis_index("c")
        b0 = wid * _BW

        pltpu.sync_copy(idx_hbm.at[pl.ds(b0 * seq, _BW * seq)], xb_v)

        def gather_start(c, b):
            pltpu.async_copy(table_hbm.at[xb_v.at[pl.ds(c * _K, _K)]],
                             rows_v.at[b], g_sem.at[b])

        def gather_wait(b):
            pltpu.make_async_copy(table_hbm.at[xb_v.at[pl.ds(0, _K)]],
                                  rows_v.at[b], g_sem.at[b]).wait()

        def scatter_start(c, b):
            pltpu.async_copy(rows_v.at[b],
                             out_hbm.at[pl.ds(b0 * seq + c * _K, _K)],
                             s_sem.at[b])

        def scatter_wait(b):
            pltpu.make_async_copy(rows_v.at[b], out_hbm.at[pl.ds(0, _K)],
                                  s_sem.at[b]).wait()

        for b in range(_PF):
            gather_start(b, b)

        def group(g, carry):
            for b in range(_NBUF):
                c = g * _NBUF + b
                gather_wait(b)

                def sc_body(r0, c2):
                    for ur in range(4):
                        r = r0 * 4 + ur
                        for u in range(_D // 16):
                            rows_v[b, r, pl.ds(u * 16, 16)] = (
                                rows_v[b, r, pl.ds(u * 16, 16)] * _SCALE
                            )
                    return c2

                lax.fori_loop(0, _K // 4, sc_body, 0)
                scatter_start(c, b)

                cp = c + _PF
                bp = (b + _PF) % _NBUF

                @pl.when(cp < n_chunks)
                def _():
                    @pl.when(cp >= _NBUF)
                    def _():
                        scatter_wait(bp)

                    gather_start(cp, bp)

            return carry

        lax.fori_loop(0, n_chunks // _NBUF, group, 0)

        for b in range(_NBUF):
            scatter_wait(b)

    return gather_scale


def kernel(x, embedding):
    batch, seq = x.shape
    idx = x.reshape(-1).astype(jnp.int32)
    tab_p = jnp.pad(embedding, ((0, 0), (0, 128 - _D)))
    out_p = _build(batch, seq)(idx, tab_p)
    return out_p.reshape(batch, seq, 128)[:, :, :_D]
